# initial kernel scaffold (unmeasured)
import jax
import jax.numpy as jnp
from jax import lax
from jax.experimental import pallas as pl
from jax.experimental.pallas import tpu as pltpu


def kernel(
    x,
):
    def body(*refs):
        pass

    out_shape = jax.ShapeDtypeStruct(..., jnp.float32)
    return pl.pallas_call(body, out_shape=out_shape)(...)


# baseline (device time: 164395 ns/iter reference)
import jax
import jax.numpy as jnp
from jax import lax
from jax.experimental import pallas as pl
from jax.experimental.pallas import tpu as pltpu

BM = 512


def kernel(x):
    M, N = x.shape
    n_chunks = M // BM

    def body(x_hbm, out_hbm, vbuf, obuf, trow, brow, recv_row,
             colblk, send_col, recv_col,
             send_sems, recv_sems, sem_in, sem_col, sem_out):
        sx = lax.axis_index("x")
        sy = lax.axis_index("y")

        barrier = pltpu.get_barrier_semaphore()
        pl.semaphore_signal(barrier, inc=1, device_id=(1 - sx, sy),
                            device_id_type=pl.DeviceIdType.MESH)
        pl.semaphore_signal(barrier, inc=1, device_id=(sx, 1 - sy),
                            device_id_type=pl.DeviceIdType.MESH)
        pl.semaphore_wait(barrier, 2)

        row_idx = jnp.where(sx == 0, M - 1, 0)
        rdma_row = pltpu.make_async_remote_copy(
            src_ref=x_hbm.at[pl.ds(row_idx, 1), :],
            dst_ref=recv_row,
            send_sem=send_sems.at[0],
            recv_sem=recv_sems.at[0],
            device_id=(1 - sx, sy),
            device_id_type=pl.DeviceIdType.MESH,
        )
        rdma_row.start()

        @pl.when(sy == 0)
        def _():
            pltpu.make_async_copy(
                x_hbm.at[:, pl.ds(N - 128, 128)], colblk, sem_col).start()

        @pl.when(sy == 1)
        def _():
            pltpu.make_async_copy(
                x_hbm.at[:, pl.ds(0, 128)], colblk, sem_col).start()

        pltpu.make_async_copy(colblk, colblk, sem_col).wait()

        @pl.when(sy == 0)
        def _():
            send_col[...] = colblk[:, 127:128]

        @pl.when(sy == 1)
        def _():
            send_col[...] = colblk[:, 0:1]

        rdma_col = pltpu.make_async_remote_copy(
            src_ref=send_col,
            dst_ref=recv_col,
            send_sem=send_sems.at[1],
            recv_sem=recv_sems.at[1],
            device_id=(sx, 1 - sy),
            device_id_type=pl.DeviceIdType.MESH,
        )
        rdma_col.start()
        rdma_row.wait()
        rdma_col.wait()

        def chunk(c, _):
            r0 = pl.multiple_of(c * BM, BM)
            cp_main = pltpu.make_async_copy(
                x_hbm.at[pl.ds(r0, BM), :], vbuf, sem_in)
            cp_main.start()

            @pl.when(c > 0)
            def _():
                pltpu.make_async_copy(
                    x_hbm.at[pl.ds(r0 - 1, 1), :], trow, sem_in).start()

            @pl.when(c == 0)
            def _():
                pltpu.make_async_copy(recv_row, trow, sem_in).start()

            @pl.when(c < n_chunks - 1)
            def _():
                pltpu.make_async_copy(
                    x_hbm.at[pl.ds(r0 + BM, 1), :], brow, sem_in).start()

            @pl.when(c == n_chunks - 1)
            def _():
                pltpu.make_async_copy(recv_row, brow, sem_in).start()

            cp_main.wait()
            pltpu.make_async_copy(trow, trow, sem_in).wait()
            pltpu.make_async_copy(brow, brow, sem_in).wait()

            vx = vbuf[...]
            north = jnp.concatenate([trow[...], vx[:BM - 1, :]], axis=0)
            south = jnp.concatenate([vx[1:, :], brow[...]], axis=0)
            hcol = recv_col[pl.ds(r0, BM), :]
            west = jnp.concatenate([hcol, vx[:, :N - 1]], axis=1)
            east = jnp.concatenate([vx[:, 1:], hcol], axis=1)
            obuf[...] = 0.5 * vx + 0.125 * (north + south + west + east)

            @pl.when(sy == 0)
            def _():
                obuf[:, 0:1] = vx[:, 0:1]

            @pl.when(sy == 1)
            def _():
                obuf[:, N - 1:N] = vx[:, N - 1:N]

            @pl.when((c == 0) & (sx == 0))
            def _():
                obuf[0:1, :] = vx[0:1, :]

            @pl.when((c == n_chunks - 1) & (sx == 1))
            def _():
                obuf[BM - 1:BM, :] = vx[BM - 1:BM, :]

            cp_out = pltpu.make_async_copy(
                obuf, out_hbm.at[pl.ds(r0, BM), :], sem_out)
            cp_out.start()
            cp_out.wait()
            return 0

        lax.fori_loop(0, n_chunks, chunk, 0)

    return pl.pallas_call(
        body,
        out_shape=jax.ShapeDtypeStruct((M, N), jnp.float32),
        in_specs=[pl.BlockSpec(memory_space=pl.ANY)],
        out_specs=pl.BlockSpec(memory_space=pl.ANY),
        scratch_shapes=[
            pltpu.VMEM((BM, N), jnp.float32),
            pltpu.VMEM((BM, N), jnp.float32),
            pltpu.VMEM((1, N), jnp.float32),
            pltpu.VMEM((1, N), jnp.float32),
            pltpu.VMEM((1, N), jnp.float32),
            pltpu.VMEM((M, 128), jnp.float32),
            pltpu.VMEM((M, 1), jnp.float32),
            pltpu.VMEM((M, 1), jnp.float32),
            pltpu.SemaphoreType.DMA((2,)),
            pltpu.SemaphoreType.DMA((2,)),
            pltpu.SemaphoreType.DMA,
            pltpu.SemaphoreType.DMA,
            pltpu.SemaphoreType.DMA,
        ],
        compiler_params=pltpu.CompilerParams(
            collective_id=0, vmem_limit_bytes=100 * 1024 * 1024),
    )(x)


# device time: 122904 ns/iter; 1.3376x vs baseline; 1.3376x over previous
import jax
import jax.numpy as jnp
from jax import lax
from jax.experimental import pallas as pl
from jax.experimental.pallas import tpu as pltpu

BM = 512


def kernel(x):
    M, N = x.shape
    n_chunks = M // BM

    def body(x_hbm, out_hbm, vbuf, obuf, trow, brow, recv_row,
             colblk, send_col, recv_col,
             send_sems, recv_sems, sem_in, sem_out, sem_col):
        sx = lax.axis_index("x")
        sy = lax.axis_index("y")

        barrier = pltpu.get_barrier_semaphore()
        pl.semaphore_signal(barrier, inc=1, device_id=(1 - sx, sy),
                            device_id_type=pl.DeviceIdType.MESH)
        pl.semaphore_signal(barrier, inc=1, device_id=(sx, 1 - sy),
                            device_id_type=pl.DeviceIdType.MESH)
        pl.semaphore_wait(barrier, 2)

        row_idx = jnp.where(sx == 0, M - 1, 0)
        rdma_row = pltpu.make_async_remote_copy(
            src_ref=x_hbm.at[pl.ds(row_idx, 1), :],
            dst_ref=recv_row,
            send_sem=send_sems.at[0],
            recv_sem=recv_sems.at[0],
            device_id=(1 - sx, sy),
            device_id_type=pl.DeviceIdType.MESH,
        )
        rdma_row.start()

        @pl.when(sy == 0)
        def _():
            pltpu.make_async_copy(
                x_hbm.at[:, pl.ds(N - 128, 128)], colblk, sem_col).start()

        @pl.when(sy == 1)
        def _():
            pltpu.make_async_copy(
                x_hbm.at[:, pl.ds(0, 128)], colblk, sem_col).start()

        pltpu.make_async_copy(colblk, colblk, sem_col).wait()

        @pl.when(sy == 0)
        def _():
            send_col[...] = colblk[:, 127:128]

        @pl.when(sy == 1)
        def _():
            send_col[...] = colblk[:, 0:1]

        rdma_col = pltpu.make_async_remote_copy(
            src_ref=send_col,
            dst_ref=recv_col,
            send_sem=send_sems.at[1],
            recv_sem=recv_sems.at[1],
            device_id=(sx, 1 - sy),
            device_id_type=pl.DeviceIdType.MESH,
        )
        rdma_col.start()
        rdma_row.wait()
        rdma_col.wait()

        def start_in(c, slot):
            r0 = pl.multiple_of(c * BM, BM)
            pltpu.make_async_copy(
                x_hbm.at[pl.ds(r0, BM), :], vbuf.at[slot], sem_in.at[slot]
            ).start()

            @pl.when(c > 0)
            def _():
                pltpu.make_async_copy(
                    x_hbm.at[pl.ds(r0 - 1, 1), :], trow.at[slot],
                    sem_in.at[slot]).start()

            @pl.when(c == 0)
            def _():
                pltpu.make_async_copy(
                    recv_row, trow.at[slot], sem_in.at[slot]).start()

            @pl.when(c < n_chunks - 1)
            def _():
                pltpu.make_async_copy(
                    x_hbm.at[pl.ds(r0 + BM, 1), :], brow.at[slot],
                    sem_in.at[slot]).start()

            @pl.when(c == n_chunks - 1)
            def _():
                pltpu.make_async_copy(
                    recv_row, brow.at[slot], sem_in.at[slot]).start()

        start_in(0, 0)

        def chunk(c, _):
            slot = lax.rem(c, 2)

            @pl.when(c < n_chunks - 1)
            def _():
                start_in(c + 1, lax.rem(c + 1, 2))

            pltpu.make_async_copy(
                vbuf.at[slot], vbuf.at[slot], sem_in.at[slot]).wait()
            pltpu.make_async_copy(
                trow.at[slot], trow.at[slot], sem_in.at[slot]).wait()
            pltpu.make_async_copy(
                brow.at[slot], brow.at[slot], sem_in.at[slot]).wait()

            @pl.when(c >= 2)
            def _():
                pltpu.make_async_copy(
                    obuf.at[slot], obuf.at[slot], sem_out.at[slot]).wait()

            r0 = pl.multiple_of(c * BM, BM)
            vx = vbuf[slot]
            north = jnp.concatenate([trow[slot], vx[:BM - 1, :]], axis=0)
            south = jnp.concatenate([vx[1:, :], brow[slot]], axis=0)
            hcol = recv_col[pl.ds(r0, BM), :]
            west = jnp.concatenate([hcol, vx[:, :N - 1]], axis=1)
            east = jnp.concatenate([vx[:, 1:], hcol], axis=1)
            obuf[slot] = 0.5 * vx + 0.125 * (north + south + west + east)

            @pl.when(sy == 0)
            def _():
                obuf[slot, :, 0:1] = vx[:, 0:1]

            @pl.when(sy == 1)
            def _():
                obuf[slot, :, N - 1:N] = vx[:, N - 1:N]

            @pl.when((c == 0) & (sx == 0))
            def _():
                obuf[slot, 0:1, :] = vx[0:1, :]

            @pl.when((c == n_chunks - 1) & (sx == 1))
            def _():
                obuf[slot, BM - 1:BM, :] = vx[BM - 1:BM, :]

            pltpu.make_async_copy(
                obuf.at[slot], out_hbm.at[pl.ds(r0, BM), :], sem_out.at[slot]
            ).start()
            return 0

        lax.fori_loop(0, n_chunks, chunk, 0)

        pltpu.make_async_copy(obuf.at[0], obuf.at[0], sem_out.at[0]).wait()
        pltpu.make_async_copy(obuf.at[1], obuf.at[1], sem_out.at[1]).wait()

    return pl.pallas_call(
        body,
        out_shape=jax.ShapeDtypeStruct((M, N), jnp.float32),
        in_specs=[pl.BlockSpec(memory_space=pl.ANY)],
        out_specs=pl.BlockSpec(memory_space=pl.ANY),
        scratch_shapes=[
            pltpu.VMEM((2, BM, N), jnp.float32),
            pltpu.VMEM((2, BM, N), jnp.float32),
            pltpu.VMEM((2, 1, N), jnp.float32),
            pltpu.VMEM((2, 1, N), jnp.float32),
            pltpu.VMEM((1, N), jnp.float32),
            pltpu.VMEM((M, 128), jnp.float32),
            pltpu.VMEM((M, 1), jnp.float32),
            pltpu.VMEM((M, 1), jnp.float32),
            pltpu.SemaphoreType.DMA((2,)),
            pltpu.SemaphoreType.DMA((2,)),
            pltpu.SemaphoreType.DMA((2,)),
            pltpu.SemaphoreType.DMA((2,)),
            pltpu.SemaphoreType.DMA,
        ],
        compiler_params=pltpu.CompilerParams(
            collective_id=0, vmem_limit_bytes=100 * 1024 * 1024),
    )(x)
